# Initial kernel scaffold; baseline (speedup 1.0000x reference)
import jax
import jax.numpy as jnp
from jax.experimental import pallas as pl


def kernel(x_p, edge_index_p, x_l, edge_index_l, bfs_init, Wp1, bp1, Wp2, bp2, Wl1, bl1, Wl2, bl2, Wd1, bd1, Wd2, bd2, Wf, bf):
    raise NotImplementedError("write your pallas kernel here")


# speedup vs baseline: 7.9253x; 7.9253x over previous
import functools

import jax
import jax.numpy as jnp
from jax import lax
from jax.experimental import pallas as pl
from jax.experimental.pallas import tpu as pltpu
from jax.experimental.pallas import tpu_sc as plsc

N = 10000
NPAD = 10240
E = 320000
EPAD = 327680
EROWS = 2560
DUMP = 10008
RPT = NPAD // 16


def _sc_mesh():
    return plsc.VectorSubcoreMesh(core_axis_name="c", subcore_axis_name="s")


def _sc_deg_both(dst_both, ones128, zeros128):
    rows_per_tile = EROWS // 16

    @functools.partial(
        pl.kernel,
        out_type=jax.ShapeDtypeStruct((2, NPAD, 128), jnp.float32),
        mesh=_sc_mesh(),
        scratch_types=[
            pltpu.VMEM((rows_per_tile, 128), jnp.int32),
            pltpu.VMEM((128, 128), jnp.float32),
            pltpu.VMEM_SHARED((NPAD, 128), jnp.float32),
        ],
    )
    def k(dst_hbm, ones_hbm, zeros_hbm, out_hbm, dst_v, ones_v, acc):
        c = lax.axis_index("c")
        s = lax.axis_index("s")
        pltpu.sync_copy(zeros_hbm, acc.at[pl.ds(s * RPT, RPT), :])
        pltpu.sync_copy(ones_hbm, ones_v)
        pltpu.sync_copy(dst_hbm.at[c, pl.ds(s * rows_per_tile, rows_per_tile), :], dst_v)
        plsc.subcore_barrier()

        def body(j, carry):
            pltpu.sync_copy(ones_v, acc.at[dst_v.at[j]], add=True)
            return carry

        lax.fori_loop(0, rows_per_tile, body, 0)
        plsc.subcore_barrier()
        pltpu.sync_copy(acc.at[pl.ds(s * RPT, RPT), :],
                        out_hbm.at[c, pl.ds(s * RPT, RPT), :])

    return k(dst_both, ones128, zeros128)


def _sc_scatter_partial(u_pad, src2d, dst2d, zeros, w):
    rows_per_tile = EROWS // 32

    @functools.partial(
        pl.kernel,
        out_type=jax.ShapeDtypeStruct((2, NPAD, w), jnp.float32),
        mesh=_sc_mesh(),
        scratch_types=[
            pltpu.VMEM((rows_per_tile, 128), jnp.int32),
            pltpu.VMEM((rows_per_tile, 128), jnp.int32),
            pltpu.VMEM((128, w), jnp.float32),
            pltpu.VMEM_SHARED((NPAD, w), jnp.float32),
            pltpu.SemaphoreType.DMA,
        ],
    )
    def k(u_hbm, src_hbm, dst_hbm, zeros_hbm, out_hbm, src_v, dst_v, rows_v, acc, sem):
        c = lax.axis_index("c")
        s = lax.axis_index("s")
        base = (c * 16 + s) * rows_per_tile
        pltpu.sync_copy(zeros_hbm, acc.at[pl.ds(s * RPT, RPT), :])
        pltpu.sync_copy(src_hbm.at[pl.ds(base, rows_per_tile), :], src_v)
        pltpu.sync_copy(dst_hbm.at[pl.ds(base, rows_per_tile), :], dst_v)
        plsc.subcore_barrier()

        def body(j, carry):
            pltpu.async_copy(u_hbm.at[src_v.at[j]], rows_v, sem).wait()
            pltpu.sync_copy(rows_v, acc.at[dst_v.at[j]], add=True)
            return carry

        lax.fori_loop(0, rows_per_tile, body, 0)
        plsc.subcore_barrier()
        pltpu.sync_copy(acc.at[pl.ds(s * RPT, RPT), :],
                        out_hbm.at[c, pl.ds(s * RPT, RPT), :])

    return k(u_pad, src2d, dst2d, zeros)


_TCB = 2560


def _tc_prep(degp16, degl16, xp_pad, xl16, Wl2, Wfp, bl2r, bfm_base):
    grid = NPAD // _TCB

    def body(degp_ref, degl_ref, xp_ref, xl_ref, wl2_ref, wfp_ref, bl2_ref,
             bfm_ref, u1p_ref, u1l_ref, invp_ref, invl_ref, wfold_ref, bfmo_ref):
        invp = lax.rsqrt(degp_ref[:, 0:1] + 1.0)
        invl = lax.rsqrt(degl_ref[:, 0:1] + 1.0)
        u1p_ref[...] = xp_ref[...] * invp
        u1l_ref[...] = jnp.zeros_like(u1l_ref)
        u1l_ref[:, 0:16] = xl_ref[...] * invl
        invp_ref[...] = jnp.broadcast_to(invp, invp_ref.shape)
        invl_ref[...] = jnp.broadcast_to(invl, invl_ref.shape)

        @pl.when(pl.program_id(0) == 0)
        def _():
            wfold_ref[...] = jnp.dot(wl2_ref[...], wfp_ref[...],
                                     preferred_element_type=jnp.float32)
            bfmo_ref[...] = jnp.dot(bl2_ref[...], wfp_ref[...],
                                    preferred_element_type=jnp.float32) + bfm_ref[...]

    return pl.pallas_call(
        body,
        grid=(grid,),
        in_specs=[
            pl.BlockSpec((_TCB, 128), lambda i: (i, 0)),
            pl.BlockSpec((_TCB, 128), lambda i: (i, 0)),
            pl.BlockSpec((_TCB, 128), lambda i: (i, 0)),
            pl.BlockSpec((_TCB, 16), lambda i: (i, 0)),
            pl.BlockSpec((128, 128), lambda i: (0, 0)),
            pl.BlockSpec((128, 16), lambda i: (0, 0)),
            pl.BlockSpec((1, 128), lambda i: (0, 0)),
            pl.BlockSpec((1, 16), lambda i: (0, 0)),
        ],
        out_specs=[
            pl.BlockSpec((_TCB, 128), lambda i: (i, 0)),
            pl.BlockSpec((_TCB, 128), lambda i: (i, 0)),
            pl.BlockSpec((_TCB, 8), lambda i: (i, 0)),
            pl.BlockSpec((_TCB, 8), lambda i: (i, 0)),
            pl.BlockSpec((128, 16), lambda i: (0, 0)),
            pl.BlockSpec((1, 16), lambda i: (0, 0)),
        ],
        out_shape=[
            jax.ShapeDtypeStruct((NPAD, 128), jnp.float32),
            jax.ShapeDtypeStruct((NPAD, 128), jnp.float32),
            jax.ShapeDtypeStruct((NPAD, 8), jnp.float32),
            jax.ShapeDtypeStruct((NPAD, 8), jnp.float32),
            jax.ShapeDtypeStruct((128, 16), jnp.float32),
            jax.ShapeDtypeStruct((1, 16), jnp.float32),
        ],
    )(degp16, degl16, xp_pad, xl16, Wl2, Wfp, bl2r, bfm_base)


def _tc_layer1(Sp, u1, inv8, W1, b1r, kdim):
    grid = NPAD // _TCB

    def body(s0_ref, s1_ref, u1_ref, inv_ref, w_ref, b_ref, u2_ref):
        inv = inv_ref[:, 0:1]
        agg = inv * (s0_ref[...] + s1_ref[...] + u1_ref[...])
        h = jnp.maximum(jnp.dot(agg, w_ref[...],
                                preferred_element_type=jnp.float32) + b_ref[...], 0.0)
        u2_ref[...] = h * inv

    return pl.pallas_call(
        body,
        grid=(grid,),
        in_specs=[
            pl.BlockSpec((_TCB, kdim), lambda i: (i, 0)),
            pl.BlockSpec((_TCB, kdim), lambda i: (i, 0)),
            pl.BlockSpec((_TCB, kdim), lambda i: (i, 0)),
            pl.BlockSpec((_TCB, 8), lambda i: (i, 0)),
            pl.BlockSpec((kdim, 128), lambda i: (0, 0)),
            pl.BlockSpec((1, 128), lambda i: (0, 0)),
        ],
        out_specs=pl.BlockSpec((_TCB, 128), lambda i: (i, 0)),
        out_shape=jax.ShapeDtypeStruct((NPAD, 128), jnp.float32),
    )(Sp[0], Sp[1], u1, inv8, W1, b1r)


_TCB3 = 2000


def _tc_heads(S2p, u2p, invp8, S2l, u2l, invl8, labv16, wfold, bfm):
    grid = N // _TCB3

    def body(sp0_ref, sp1_ref, up_ref, ip_ref, sl0_ref, sl1_ref, ul_ref,
             il_ref, lab_ref, wf_ref, bfm_ref,
             saggp_ref, saggl_ref, logp_ref, slab_ref,
             accp, accl, acclp, acclab):
        i = pl.program_id(0)

        @pl.when(i == 0)
        def _():
            accp[...] = jnp.zeros_like(accp)
            accl[...] = jnp.zeros_like(accl)
            acclp[...] = jnp.zeros_like(acclp)
            acclab[...] = jnp.zeros_like(acclab)

        aggp = ip_ref[:, 0:1] * (sp0_ref[...] + sp1_ref[...] + up_ref[...])
        aggl = il_ref[:, 0:1] * (sl0_ref[...] + sl1_ref[...] + ul_ref[...])
        accp[...] += jnp.sum(aggp, axis=0, keepdims=True)
        accl[...] += jnp.sum(aggl, axis=0, keepdims=True)
        lab = lab_ref[...]
        acclab[...] += jnp.sum(lab, axis=0, keepdims=True)
        logits = jnp.dot(aggl, wf_ref[...],
                         preferred_element_type=jnp.float32) + bfm_ref[...]
        m = jnp.max(logits, axis=1, keepdims=True)
        e = jnp.exp(logits - m)
        probs = e / jnp.sum(e, axis=1, keepdims=True)
        p = jnp.sum(probs * lab, axis=1, keepdims=True)
        acclp[...] += jnp.sum(jnp.log(p), keepdims=True).reshape(1, 1)

        @pl.when(i == grid - 1)
        def _():
            saggp_ref[...] = accp[...]
            saggl_ref[...] = accl[...]
            logp_ref[...] = acclp[...]
            slab_ref[...] = acclab[...]

    return pl.pallas_call(
        body,
        grid=(grid,),
        in_specs=[
            pl.BlockSpec((_TCB3, 128), lambda i: (i, 0)),
            pl.BlockSpec((_TCB3, 128), lambda i: (i, 0)),
            pl.BlockSpec((_TCB3, 128), lambda i: (i, 0)),
            pl.BlockSpec((_TCB3, 8), lambda i: (i, 0)),
            pl.BlockSpec((_TCB3, 128), lambda i: (i, 0)),
            pl.BlockSpec((_TCB3, 128), lambda i: (i, 0)),
            pl.BlockSpec((_TCB3, 128), lambda i: (i, 0)),
            pl.BlockSpec((_TCB3, 8), lambda i: (i, 0)),
            pl.BlockSpec((_TCB3, 16), lambda i: (i, 0)),
            pl.BlockSpec((128, 16), lambda i: (0, 0)),
            pl.BlockSpec((1, 16), lambda i: (0, 0)),
        ],
        out_specs=[
            pl.BlockSpec((1, 128), lambda i: (0, 0)),
            pl.BlockSpec((1, 128), lambda i: (0, 0)),
            pl.BlockSpec((1, 1), lambda i: (0, 0)),
            pl.BlockSpec((1, 16), lambda i: (0, 0)),
        ],
        out_shape=[
            jax.ShapeDtypeStruct((1, 128), jnp.float32),
            jax.ShapeDtypeStruct((1, 128), jnp.float32),
            jax.ShapeDtypeStruct((1, 1), jnp.float32),
            jax.ShapeDtypeStruct((1, 16), jnp.float32),
        ],
        scratch_shapes=[
            pltpu.VMEM((1, 128), jnp.float32),
            pltpu.VMEM((1, 128), jnp.float32),
            pltpu.VMEM((1, 1), jnp.float32),
            pltpu.VMEM((1, 16), jnp.float32),
        ],
    )(S2p[0], S2p[1], u2p, invp8, S2l[0], S2l[1], u2l, invl8,
      labv16, wfold, bfm)


def _tc_decoder(s0d0, labv2, Wd1p, bd1r, Wd2, bd2r, Wp2, bp2r, Wl2, bl2r,
                saggp, saggl, slab):
    NV = float(N + 1)
    ISQ2 = 0.7071067811865476

    def body(idx_ref, lab_ref, wd1_ref, bd1_ref, wd2_ref, bd2_ref, wp2_ref,
             bp2_ref, wl2_ref, bl2_ref, saggp_ref, saggl_ref, slab_ref,
             zv_ref, zp_ref, hi_ref, ht_ref):
        s0 = idx_ref[0]
        d0 = idx_ref[1]
        rows = lax.broadcasted_iota(jnp.int32, (NPAD, 1), 0)
        oh = (rows == d0).astype(jnp.float32)
        oh_s = (rows == s0).astype(jnp.float32)
        valid = (rows < N + 1).astype(jnp.float32)
        inv = 1.0 - oh * (1.0 - ISQ2)
        lab = lab_ref[...]
        u = lab * inv
        u_s0 = jnp.sum(u * oh_s, axis=0, keepdims=True)
        agg1 = inv * (u + oh * u_s0)
        h = jnp.maximum(jnp.dot(agg1, wd1_ref[...],
                                preferred_element_type=jnp.float32) + bd1_ref[...], 0.0)
        u2 = h * inv
        u2_s0 = jnp.sum(u2 * oh_s, axis=0, keepdims=True)
        agg2 = inv * (u2 + oh * u2_s0)
        zv = jnp.dot(agg2, wd2_ref[...],
                     preferred_element_type=jnp.float32) + bd2_ref[...]
        zv_ref[...] = zv
        zv_mean = jnp.sum(zv * valid, axis=0, keepdims=True) / NV
        cols = lax.broadcasted_iota(jnp.int32, (1, 16), 1)
        lab2_mean = (slab_ref[...] + (cols == 10).astype(jnp.float32)) / NV
        ht_ref[...] = jnp.concatenate([zv_mean, lab2_mean], axis=1)
        mz_l = jnp.dot(saggl_ref[...] / float(N), wl2_ref[...],
                       preferred_element_type=jnp.float32) + bl2_ref[...]
        hi_ref[...] = jnp.concatenate([mz_l, slab_ref[...] / float(N)], axis=1)
        zp_ref[...] = jnp.dot(saggp_ref[...] / float(N), wp2_ref[...],
                              preferred_element_type=jnp.float32) + bp2_ref[...]

    return pl.pallas_call(
        body,
        grid=(1,),
        in_specs=[
            pl.BlockSpec(memory_space=pltpu.SMEM),
            pl.BlockSpec((NPAD, 16), lambda i: (0, 0)),
            pl.BlockSpec((16, 128), lambda i: (0, 0)),
            pl.BlockSpec((1, 128), lambda i: (0, 0)),
            pl.BlockSpec((128, 128), lambda i: (0, 0)),
            pl.BlockSpec((1, 128), lambda i: (0, 0)),
            pl.BlockSpec((128, 128), lambda i: (0, 0)),
            pl.BlockSpec((1, 128), lambda i: (0, 0)),
            pl.BlockSpec((128, 128), lambda i: (0, 0)),
            pl.BlockSpec((1, 128), lambda i: (0, 0)),
            pl.BlockSpec((1, 128), lambda i: (0, 0)),
            pl.BlockSpec((1, 128), lambda i: (0, 0)),
            pl.BlockSpec((1, 16), lambda i: (0, 0)),
        ],
        out_specs=[
            pl.BlockSpec((NPAD, 128), lambda i: (0, 0)),
            pl.BlockSpec((1, 128), lambda i: (0, 0)),
            pl.BlockSpec((1, 144), lambda i: (0, 0)),
            pl.BlockSpec((1, 144), lambda i: (0, 0)),
        ],
        out_shape=[
            jax.ShapeDtypeStruct((NPAD, 128), jnp.float32),
            jax.ShapeDtypeStruct((1, 128), jnp.float32),
            jax.ShapeDtypeStruct((1, 144), jnp.float32),
            jax.ShapeDtypeStruct((1, 144), jnp.float32),
        ],
    )(s0d0, labv2, Wd1p, bd1r, Wd2, bd2r, Wp2, bp2r, Wl2, bl2r, saggp, saggl, slab)


def _pad_edges(ei):
    pad = jnp.full((EPAD - E,), DUMP, jnp.int32)
    src = jnp.concatenate([ei[0].astype(jnp.int32), pad]).reshape(EROWS, 128)
    dst = jnp.concatenate([ei[1].astype(jnp.int32), pad]).reshape(EROWS, 128)
    return src, dst


def kernel(x_p, edge_index_p, x_l, edge_index_l, bfs_init, Wp1, bp1, Wp2, bp2,
           Wl1, bl1, Wl2, bl2, Wd1, bd1, Wd2, bd2, Wf, bf):
    f32 = jnp.float32
    srcp, dstp = _pad_edges(edge_index_p)
    srcl, dstl = _pad_edges(edge_index_l)
    dst_both = jnp.stack([dstp, dstl])
    ones128 = jnp.ones((128, 128), f32)
    zeros128 = jnp.zeros((RPT, 128), f32)

    xp_pad = jnp.pad(x_p, ((0, NPAD - N), (0, 0)))
    xl16 = jnp.pad(x_l, ((0, NPAD - N), (0, 1)))
    labv16 = jnp.pad(x_l[:, 4:], ((0, NPAD - N), (0, 5)))
    stop16 = jnp.zeros((1, 16), f32).at[0, 10].set(1.0)
    labv2 = jnp.concatenate(
        [labv16[:N], stop16, jnp.zeros((NPAD - N - 1, 16), f32)])

    Wl1p = jnp.pad(Wl1, ((0, 113), (0, 0)))
    Wd1p = jnp.pad(Wd1, ((0, 5), (0, 0)))
    Wfp = jnp.pad(Wf, ((0, 0), (0, 5)))
    bfm_base = (jnp.pad(bf, (0, 5)).reshape(1, 16)
                + jnp.concatenate([jnp.zeros((10,), f32),
                                   jnp.full((6,), -1e9, f32)]).reshape(1, 16))

    deg = _sc_deg_both(dst_both, ones128, zeros128)

    u1p, u1l, invp8, invl8, wfold, bfm = _tc_prep(
        deg[0], deg[1], xp_pad, xl16, Wl2, Wfp, bl2.reshape(1, 128), bfm_base)

    S1p = _sc_scatter_partial(u1p, srcp, dstp, zeros128, 128)
    S1l = _sc_scatter_partial(u1l, srcl, dstl, zeros128, 128)
    u2p = _tc_layer1(S1p, u1p, invp8, Wp1, bp1.reshape(1, 128), 128)
    u2l = _tc_layer1(S1l, u1l, invl8, Wl1p, bl1.reshape(1, 128), 128)

    S2p = _sc_scatter_partial(u2p, srcp, dstp, zeros128, 128)
    S2l = _sc_scatter_partial(u2l, srcl, dstl, zeros128, 128)
    saggp, saggl, logp, slab = _tc_heads(
        S2p, u2p, invp8, S2l, u2l, invl8, labv16, wfold, bfm)

    zv_pad, zp, hi, ht = _tc_decoder(
        bfs_init.reshape(2).astype(jnp.int32), labv2, Wd1p,
        bd1.reshape(1, 128), Wd2, bd2.reshape(1, 128), Wp2,
        bp2.reshape(1, 128), Wl2, bl2.reshape(1, 128), saggp, saggl, slab)

    log_prob = logp.reshape(())
    z_pocket = zp.reshape(128)
    z_v = zv_pad[:N + 1]
    H_init = hi.reshape(144)[:139]
    H_t = ht.reshape(144)[:139]
    return (log_prob, z_pocket, z_v, H_init, H_t)


# speedup vs baseline: 7.9624x; 1.0047x over previous
import functools

import jax
import jax.numpy as jnp
from jax import lax
from jax.experimental import pallas as pl
from jax.experimental.pallas import tpu as pltpu
from jax.experimental.pallas import tpu_sc as plsc

N = 10000
NPAD = 10240
E = 320000
EPAD = 327680
EROWS = 2560
DUMP = 10008
RPT = NPAD // 16
QD = 4
EROWS64 = 5120


def _sc_mesh():
    return plsc.VectorSubcoreMesh(core_axis_name="c", subcore_axis_name="s")


def _sc_deg_both(dst_both, ones128, zeros128):
    rows_per_tile = EROWS64 // 16

    @functools.partial(
        pl.kernel,
        out_type=jax.ShapeDtypeStruct((2, NPAD, 128), jnp.float32),
        mesh=_sc_mesh(),
        scratch_types=[
            pltpu.VMEM((rows_per_tile, 64), jnp.int32),
            pltpu.VMEM((64, 128), jnp.float32),
            pltpu.VMEM_SHARED((NPAD, 128), jnp.float32),
            pltpu.SemaphoreType.DMA,
        ],
    )
    def k(dst_hbm, ones_hbm, zeros_hbm, out_hbm, dst_v, ones_v, acc, ssem):
        c = lax.axis_index("c")
        s = lax.axis_index("s")
        pltpu.sync_copy(zeros_hbm, acc.at[pl.ds(s * RPT, RPT), :])
        pltpu.sync_copy(ones_hbm, ones_v)
        pltpu.sync_copy(dst_hbm.at[c, pl.ds(s * rows_per_tile, rows_per_tile), :], dst_v)
        plsc.subcore_barrier()

        def group(g, carry):
            j0 = g * QD
            sd = [pltpu.async_copy(ones_v, acc.at[dst_v.at[j0 + q]], ssem,
                                   add=True)
                  for q in range(QD)]
            for q in range(QD):
                sd[q].wait()
            return carry

        lax.fori_loop(0, rows_per_tile // QD, group, 0)
        plsc.subcore_barrier()
        pltpu.sync_copy(acc.at[pl.ds(s * RPT, RPT), :],
                        out_hbm.at[c, pl.ds(s * RPT, RPT), :])

    return k(dst_both, ones128, zeros128)


def _sc_scatter_partial(u_pad, src2d, dst2d, zeros, w):
    rows_per_tile = EROWS64 // 32
    SG = 8 * QD

    @functools.partial(
        pl.kernel,
        out_type=jax.ShapeDtypeStruct((2, NPAD, w), jnp.float32),
        mesh=_sc_mesh(),
        scratch_types=[
            pltpu.VMEM((SG, 64), jnp.int32),
            pltpu.VMEM((SG, 64), jnp.int32),
            *[pltpu.VMEM((64, w), jnp.float32) for _ in range(QD)],
            pltpu.VMEM_SHARED((NPAD, w), jnp.float32),
            pltpu.SemaphoreType.DMA,
            pltpu.SemaphoreType.DMA,
        ],
    )
    def k(u_hbm, src_hbm, dst_hbm, zeros_hbm, out_hbm, src_v, dst_v,
          b0, b1, b2, b3, acc, gsem, ssem):
        c = lax.axis_index("c")
        s = lax.axis_index("s")
        base = (c * 16 + s) * rows_per_tile
        pltpu.sync_copy(zeros_hbm, acc.at[pl.ds(s * RPT, RPT), :])
        plsc.subcore_barrier()
        bufs = [b0, b1, b2, b3][:QD]

        def group(g, carry):
            @pl.when(g % (SG // QD) == 0)
            def _():
                sgb = base + (g // (SG // QD)) * SG
                pltpu.sync_copy(src_hbm.at[pl.ds(sgb, SG), :], src_v)
                pltpu.sync_copy(dst_hbm.at[pl.ds(sgb, SG), :], dst_v)

            lr = (g % (SG // QD)) * QD
            gd = [pltpu.async_copy(u_hbm.at[src_v.at[lr + q]], bufs[q], gsem)
                  for q in range(QD)]
            for q in range(QD):
                gd[q].wait()
            sd = [pltpu.async_copy(bufs[q], acc.at[dst_v.at[lr + q]], ssem,
                                   add=True)
                  for q in range(QD)]
            for q in range(QD):
                sd[q].wait()
            return carry

        lax.fori_loop(0, rows_per_tile // QD, group, 0)
        plsc.subcore_barrier()
        pltpu.sync_copy(acc.at[pl.ds(s * RPT, RPT), :],
                        out_hbm.at[c, pl.ds(s * RPT, RPT), :])

    return k(u_pad, src2d, dst2d, zeros)


_TCB = 2560


def _tc_prep(degp16, degl16, xp_pad, xl16, Wl2, Wfp, bl2r, bfm_base):
    grid = NPAD // _TCB

    def body(degp_ref, degl_ref, xp_ref, xl_ref, wl2_ref, wfp_ref, bl2_ref,
             bfm_ref, u1p_ref, u1l_ref, invp_ref, invl_ref, wfold_ref, bfmo_ref):
        invp = lax.rsqrt(degp_ref[:, 0:1] + 1.0)
        invl = lax.rsqrt(degl_ref[:, 0:1] + 1.0)
        u1p_ref[...] = xp_ref[...] * invp
        u1l_ref[...] = jnp.zeros_like(u1l_ref)
        u1l_ref[:, 0:16] = xl_ref[...] * invl
        invp_ref[...] = jnp.broadcast_to(invp, invp_ref.shape)
        invl_ref[...] = jnp.broadcast_to(invl, invl_ref.shape)

        @pl.when(pl.program_id(0) == 0)
        def _():
            wfold_ref[...] = jnp.dot(wl2_ref[...], wfp_ref[...],
                                     preferred_element_type=jnp.float32)
            bfmo_ref[...] = jnp.dot(bl2_ref[...], wfp_ref[...],
                                    preferred_element_type=jnp.float32) + bfm_ref[...]

    return pl.pallas_call(
        body,
        grid=(grid,),
        in_specs=[
            pl.BlockSpec((_TCB, 128), lambda i: (i, 0)),
            pl.BlockSpec((_TCB, 128), lambda i: (i, 0)),
            pl.BlockSpec((_TCB, 128), lambda i: (i, 0)),
            pl.BlockSpec((_TCB, 16), lambda i: (i, 0)),
            pl.BlockSpec((128, 128), lambda i: (0, 0)),
            pl.BlockSpec((128, 16), lambda i: (0, 0)),
            pl.BlockSpec((1, 128), lambda i: (0, 0)),
            pl.BlockSpec((1, 16), lambda i: (0, 0)),
        ],
        out_specs=[
            pl.BlockSpec((_TCB, 128), lambda i: (i, 0)),
            pl.BlockSpec((_TCB, 128), lambda i: (i, 0)),
            pl.BlockSpec((_TCB, 8), lambda i: (i, 0)),
            pl.BlockSpec((_TCB, 8), lambda i: (i, 0)),
            pl.BlockSpec((128, 16), lambda i: (0, 0)),
            pl.BlockSpec((1, 16), lambda i: (0, 0)),
        ],
        out_shape=[
            jax.ShapeDtypeStruct((NPAD, 128), jnp.float32),
            jax.ShapeDtypeStruct((NPAD, 128), jnp.float32),
            jax.ShapeDtypeStruct((NPAD, 8), jnp.float32),
            jax.ShapeDtypeStruct((NPAD, 8), jnp.float32),
            jax.ShapeDtypeStruct((128, 16), jnp.float32),
            jax.ShapeDtypeStruct((1, 16), jnp.float32),
        ],
    )(degp16, degl16, xp_pad, xl16, Wl2, Wfp, bl2r, bfm_base)


def _tc_layer1(Sp, u1, inv8, W1, b1r, kdim):
    grid = NPAD // _TCB

    def body(s0_ref, s1_ref, u1_ref, inv_ref, w_ref, b_ref, u2_ref):
        inv = inv_ref[:, 0:1]
        agg = inv * (s0_ref[...] + s1_ref[...] + u1_ref[...])
        h = jnp.maximum(jnp.dot(agg, w_ref[...],
                                preferred_element_type=jnp.float32) + b_ref[...], 0.0)
        u2_ref[...] = h * inv

    return pl.pallas_call(
        body,
        grid=(grid,),
        in_specs=[
            pl.BlockSpec((_TCB, kdim), lambda i: (i, 0)),
            pl.BlockSpec((_TCB, kdim), lambda i: (i, 0)),
            pl.BlockSpec((_TCB, kdim), lambda i: (i, 0)),
            pl.BlockSpec((_TCB, 8), lambda i: (i, 0)),
            pl.BlockSpec((kdim, 128), lambda i: (0, 0)),
            pl.BlockSpec((1, 128), lambda i: (0, 0)),
        ],
        out_specs=pl.BlockSpec((_TCB, 128), lambda i: (i, 0)),
        out_shape=jax.ShapeDtypeStruct((NPAD, 128), jnp.float32),
    )(Sp[0], Sp[1], u1, inv8, W1, b1r)


_TCB3 = 2000


def _tc_heads(S2p, u2p, invp8, S2l, u2l, invl8, labv16, wfold, bfm):
    grid = N // _TCB3

    def body(sp0_ref, sp1_ref, up_ref, ip_ref, sl0_ref, sl1_ref, ul_ref,
             il_ref, lab_ref, wf_ref, bfm_ref,
             saggp_ref, saggl_ref, logp_ref, slab_ref,
             accp, accl, acclp, acclab):
        i = pl.program_id(0)

        @pl.when(i == 0)
        def _():
            accp[...] = jnp.zeros_like(accp)
            accl[...] = jnp.zeros_like(accl)
            acclp[...] = jnp.zeros_like(acclp)
            acclab[...] = jnp.zeros_like(acclab)

        aggp = ip_ref[:, 0:1] * (sp0_ref[...] + sp1_ref[...] + up_ref[...])
        aggl = il_ref[:, 0:1] * (sl0_ref[...] + sl1_ref[...] + ul_ref[...])
        accp[...] += jnp.sum(aggp, axis=0, keepdims=True)
        accl[...] += jnp.sum(aggl, axis=0, keepdims=True)
        lab = lab_ref[...]
        acclab[...] += jnp.sum(lab, axis=0, keepdims=True)
        logits = jnp.dot(aggl, wf_ref[...],
                         preferred_element_type=jnp.float32) + bfm_ref[...]
        m = jnp.max(logits, axis=1, keepdims=True)
        e = jnp.exp(logits - m)
        probs = e / jnp.sum(e, axis=1, keepdims=True)
        p = jnp.sum(probs * lab, axis=1, keepdims=True)
        acclp[...] += jnp.sum(jnp.log(p), keepdims=True).reshape(1, 1)

        @pl.when(i == grid - 1)
        def _():
            saggp_ref[...] = accp[...]
            saggl_ref[...] = accl[...]
            logp_ref[...] = acclp[...]
            slab_ref[...] = acclab[...]

    return pl.pallas_call(
        body,
        grid=(grid,),
        in_specs=[
            pl.BlockSpec((_TCB3, 128), lambda i: (i, 0)),
            pl.BlockSpec((_TCB3, 128), lambda i: (i, 0)),
            pl.BlockSpec((_TCB3, 128), lambda i: (i, 0)),
            pl.BlockSpec((_TCB3, 8), lambda i: (i, 0)),
            pl.BlockSpec((_TCB3, 128), lambda i: (i, 0)),
            pl.BlockSpec((_TCB3, 128), lambda i: (i, 0)),
            pl.BlockSpec((_TCB3, 128), lambda i: (i, 0)),
            pl.BlockSpec((_TCB3, 8), lambda i: (i, 0)),
            pl.BlockSpec((_TCB3, 16), lambda i: (i, 0)),
            pl.BlockSpec((128, 16), lambda i: (0, 0)),
            pl.BlockSpec((1, 16), lambda i: (0, 0)),
        ],
        out_specs=[
            pl.BlockSpec((1, 128), lambda i: (0, 0)),
            pl.BlockSpec((1, 128), lambda i: (0, 0)),
            pl.BlockSpec((1, 1), lambda i: (0, 0)),
            pl.BlockSpec((1, 16), lambda i: (0, 0)),
        ],
        out_shape=[
            jax.ShapeDtypeStruct((1, 128), jnp.float32),
            jax.ShapeDtypeStruct((1, 128), jnp.float32),
            jax.ShapeDtypeStruct((1, 1), jnp.float32),
            jax.ShapeDtypeStruct((1, 16), jnp.float32),
        ],
        scratch_shapes=[
            pltpu.VMEM((1, 128), jnp.float32),
            pltpu.VMEM((1, 128), jnp.float32),
            pltpu.VMEM((1, 1), jnp.float32),
            pltpu.VMEM((1, 16), jnp.float32),
        ],
    )(S2p[0], S2p[1], u2p, invp8, S2l[0], S2l[1], u2l, invl8,
      labv16, wfold, bfm)


def _tc_decoder(s0d0, labv2, Wd1p, bd1r, Wd2, bd2r, Wp2, bp2r, Wl2, bl2r,
                saggp, saggl, slab):
    NV = float(N + 1)
    ISQ2 = 0.7071067811865476

    def body(idx_ref, lab_ref, wd1_ref, bd1_ref, wd2_ref, bd2_ref, wp2_ref,
             bp2_ref, wl2_ref, bl2_ref, saggp_ref, saggl_ref, slab_ref,
             zv_ref, zp_ref, hi_ref, ht_ref):
        s0 = idx_ref[0]
        d0 = idx_ref[1]
        rows = lax.broadcasted_iota(jnp.int32, (NPAD, 1), 0)
        oh = (rows == d0).astype(jnp.float32)
        oh_s = (rows == s0).astype(jnp.float32)
        valid = (rows < N + 1).astype(jnp.float32)
        inv = 1.0 - oh * (1.0 - ISQ2)
        lab = lab_ref[...]
        u = lab * inv
        u_s0 = jnp.sum(u * oh_s, axis=0, keepdims=True)
        agg1 = inv * (u + oh * u_s0)
        h = jnp.maximum(jnp.dot(agg1, wd1_ref[...],
                                preferred_element_type=jnp.float32) + bd1_ref[...], 0.0)
        u2 = h * inv
        u2_s0 = jnp.sum(u2 * oh_s, axis=0, keepdims=True)
        agg2 = inv * (u2 + oh * u2_s0)
        zv = jnp.dot(agg2, wd2_ref[...],
                     preferred_element_type=jnp.float32) + bd2_ref[...]
        zv_ref[...] = zv
        zv_mean = jnp.sum(zv * valid, axis=0, keepdims=True) / NV
        cols = lax.broadcasted_iota(jnp.int32, (1, 16), 1)
        lab2_mean = (slab_ref[...] + (cols == 10).astype(jnp.float32)) / NV
        ht_ref[...] = jnp.concatenate([zv_mean, lab2_mean], axis=1)
        mz_l = jnp.dot(saggl_ref[...] / float(N), wl2_ref[...],
                       preferred_element_type=jnp.float32) + bl2_ref[...]
        hi_ref[...] = jnp.concatenate([mz_l, slab_ref[...] / float(N)], axis=1)
        zp_ref[...] = jnp.dot(saggp_ref[...] / float(N), wp2_ref[...],
                              preferred_element_type=jnp.float32) + bp2_ref[...]

    return pl.pallas_call(
        body,
        grid=(1,),
        in_specs=[
            pl.BlockSpec(memory_space=pltpu.SMEM),
            pl.BlockSpec((NPAD, 16), lambda i: (0, 0)),
            pl.BlockSpec((16, 128), lambda i: (0, 0)),
            pl.BlockSpec((1, 128), lambda i: (0, 0)),
            pl.BlockSpec((128, 128), lambda i: (0, 0)),
            pl.BlockSpec((1, 128), lambda i: (0, 0)),
            pl.BlockSpec((128, 128), lambda i: (0, 0)),
            pl.BlockSpec((1, 128), lambda i: (0, 0)),
            pl.BlockSpec((128, 128), lambda i: (0, 0)),
            pl.BlockSpec((1, 128), lambda i: (0, 0)),
            pl.BlockSpec((1, 128), lambda i: (0, 0)),
            pl.BlockSpec((1, 128), lambda i: (0, 0)),
            pl.BlockSpec((1, 16), lambda i: (0, 0)),
        ],
        out_specs=[
            pl.BlockSpec((NPAD, 128), lambda i: (0, 0)),
            pl.BlockSpec((1, 128), lambda i: (0, 0)),
            pl.BlockSpec((1, 144), lambda i: (0, 0)),
            pl.BlockSpec((1, 144), lambda i: (0, 0)),
        ],
        out_shape=[
            jax.ShapeDtypeStruct((NPAD, 128), jnp.float32),
            jax.ShapeDtypeStruct((1, 128), jnp.float32),
            jax.ShapeDtypeStruct((1, 144), jnp.float32),
            jax.ShapeDtypeStruct((1, 144), jnp.float32),
        ],
    )(s0d0, labv2, Wd1p, bd1r, Wd2, bd2r, Wp2, bp2r, Wl2, bl2r, saggp, saggl, slab)


def _pad_edges(ei):
    pad = jnp.full((EPAD - E,), DUMP, jnp.int32)
    src = jnp.concatenate([ei[0].astype(jnp.int32), pad]).reshape(EROWS64, 64)
    dst = jnp.concatenate([ei[1].astype(jnp.int32), pad]).reshape(EROWS64, 64)
    return src, dst


def kernel(x_p, edge_index_p, x_l, edge_index_l, bfs_init, Wp1, bp1, Wp2, bp2,
           Wl1, bl1, Wl2, bl2, Wd1, bd1, Wd2, bd2, Wf, bf):
    f32 = jnp.float32
    srcp, dstp = _pad_edges(edge_index_p)
    srcl, dstl = _pad_edges(edge_index_l)
    dst_both = jnp.stack([dstp, dstl])
    ones128 = jnp.ones((64, 128), f32)
    zeros128 = jnp.zeros((RPT, 128), f32)

    xp_pad = jnp.pad(x_p, ((0, NPAD - N), (0, 0)))
    xl16 = jnp.pad(x_l, ((0, NPAD - N), (0, 1)))
    labv16 = jnp.pad(x_l[:, 4:], ((0, NPAD - N), (0, 5)))
    stop16 = jnp.zeros((1, 16), f32).at[0, 10].set(1.0)
    labv2 = jnp.concatenate(
        [labv16[:N], stop16, jnp.zeros((NPAD - N - 1, 16), f32)])

    Wl1p = jnp.pad(Wl1, ((0, 113), (0, 0)))
    Wd1p = jnp.pad(Wd1, ((0, 5), (0, 0)))
    Wfp = jnp.pad(Wf, ((0, 0), (0, 5)))
    bfm_base = (jnp.pad(bf, (0, 5)).reshape(1, 16)
                + jnp.concatenate([jnp.zeros((10,), f32),
                                   jnp.full((6,), -1e9, f32)]).reshape(1, 16))

    deg = _sc_deg_both(dst_both, ones128, zeros128)

    u1p, u1l, invp8, invl8, wfold, bfm = _tc_prep(
        deg[0], deg[1], xp_pad, xl16, Wl2, Wfp, bl2.reshape(1, 128), bfm_base)

    S1p = _sc_scatter_partial(u1p, srcp, dstp, zeros128, 128)
    S1l = _sc_scatter_partial(u1l, srcl, dstl, zeros128, 128)
    u2p = _tc_layer1(S1p, u1p, invp8, Wp1, bp1.reshape(1, 128), 128)
    u2l = _tc_layer1(S1l, u1l, invl8, Wl1p, bl1.reshape(1, 128), 128)

    S2p = _sc_scatter_partial(u2p, srcp, dstp, zeros128, 128)
    S2l = _sc_scatter_partial(u2l, srcl, dstl, zeros128, 128)
    saggp, saggl, logp, slab = _tc_heads(
        S2p, u2p, invp8, S2l, u2l, invl8, labv16, wfold, bfm)

    zv_pad, zp, hi, ht = _tc_decoder(
        bfs_init.reshape(2).astype(jnp.int32), labv2, Wd1p,
        bd1.reshape(1, 128), Wd2, bd2.reshape(1, 128), Wp2,
        bp2.reshape(1, 128), Wl2, bl2.reshape(1, 128), saggp, saggl, slab)

    log_prob = logp.reshape(())
    z_pocket = zp.reshape(128)
    z_v = zv_pad[:N + 1]
    H_init = hi.reshape(144)[:139]
    H_t = ht.reshape(144)[:139]
    return (log_prob, z_pocket, z_v, H_init, H_t)


# speedup vs baseline: 10.0145x; 1.2577x over previous
import functools

import jax
import jax.numpy as jnp
from jax import lax
from jax.experimental import pallas as pl
from jax.experimental.pallas import tpu as pltpu
from jax.experimental.pallas import tpu_sc as plsc

N = 10000
NPAD = 10240
E = 320000
EPAD = 327680
EROWS = 2560
DUMP = 10008
RPT = NPAD // 16
QD = 4
EROWS64 = 5120
SPLIT0 = 288
SPLIT1 = 32


def _sc_mesh():
    return plsc.VectorSubcoreMesh(core_axis_name="c", subcore_axis_name="s")


def _sc_deg_both(dst_both, ones128, zeros128):
    rows_per_tile = EROWS64 // 16

    @functools.partial(
        pl.kernel,
        out_type=jax.ShapeDtypeStruct((2, NPAD, 128), jnp.float32),
        mesh=_sc_mesh(),
        scratch_types=[
            pltpu.VMEM((rows_per_tile, 64), jnp.int32),
            pltpu.VMEM((64, 128), jnp.float32),
            pltpu.VMEM_SHARED((NPAD, 128), jnp.float32),
            pltpu.SemaphoreType.DMA,
        ],
    )
    def k(dst_hbm, ones_hbm, zeros_hbm, out_hbm, dst_v, ones_v, acc, ssem):
        c = lax.axis_index("c")
        s = lax.axis_index("s")
        pltpu.sync_copy(zeros_hbm, acc.at[pl.ds(s * RPT, RPT), :])
        pltpu.sync_copy(ones_hbm, ones_v)
        pltpu.sync_copy(dst_hbm.at[c, pl.ds(s * rows_per_tile, rows_per_tile), :], dst_v)
        plsc.subcore_barrier()

        def group(g, carry):
            j0 = g * QD
            sd = [pltpu.async_copy(ones_v, acc.at[dst_v.at[j0 + q]], ssem,
                                   add=True)
                  for q in range(QD)]
            for q in range(QD):
                sd[q].wait()
            return carry

        lax.fori_loop(0, rows_per_tile // QD, group, 0)
        plsc.subcore_barrier()
        pltpu.sync_copy(acc.at[pl.ds(s * RPT, RPT), :],
                        out_hbm.at[c, pl.ds(s * RPT, RPT), :])

    return k(dst_both, ones128, zeros128)


def _sc_scatter_partial(u_pad, src2d, dst2d, zeros, w):
    SG = 8 * QD

    @functools.partial(
        pl.kernel,
        out_type=jax.ShapeDtypeStruct((2, NPAD, w), jnp.float32),
        mesh=_sc_mesh(),
        scratch_types=[
            pltpu.VMEM((SG, 64), jnp.int32),
            pltpu.VMEM((SG, 64), jnp.int32),
            *[pltpu.VMEM((64, w), jnp.float32) for _ in range(QD)],
            pltpu.VMEM_SHARED((NPAD, w), jnp.float32),
            pltpu.SemaphoreType.DMA,
            pltpu.SemaphoreType.DMA,
        ],
    )
    def k(u_hbm, src_hbm, dst_hbm, zeros_hbm, out_hbm, src_v, dst_v,
          b0, b1, b2, b3, acc, gsem, ssem):
        c = lax.axis_index("c")
        s = lax.axis_index("s")
        base = jnp.where(c == 0, s * SPLIT0, 16 * SPLIT0 + s * SPLIT1)
        n_groups = jnp.where(c == 0, SPLIT0 // QD, SPLIT1 // QD)
        pltpu.sync_copy(zeros_hbm, acc.at[pl.ds(s * RPT, RPT), :])
        plsc.subcore_barrier()
        bufs = [b0, b1, b2, b3][:QD]

        def group(g, carry):
            @pl.when(g % (SG // QD) == 0)
            def _():
                sgb = base + (g // (SG // QD)) * SG
                pltpu.sync_copy(src_hbm.at[pl.ds(sgb, SG), :], src_v)
                pltpu.sync_copy(dst_hbm.at[pl.ds(sgb, SG), :], dst_v)

            lr = (g % (SG // QD)) * QD
            gd = [pltpu.async_copy(u_hbm.at[src_v.at[lr + q]], bufs[q], gsem)
                  for q in range(QD)]
            for q in range(QD):
                gd[q].wait()
            sd = [pltpu.async_copy(bufs[q], acc.at[dst_v.at[lr + q]], ssem,
                                   add=True)
                  for q in range(QD)]
            for q in range(QD):
                sd[q].wait()
            return carry

        lax.fori_loop(0, n_groups, group, 0)
        plsc.subcore_barrier()
        pltpu.sync_copy(acc.at[pl.ds(s * RPT, RPT), :],
                        out_hbm.at[c, pl.ds(s * RPT, RPT), :])

    return k(u_pad, src2d, dst2d, zeros)


_TCB = 2560


def _tc_prep(degp16, degl16, xp_pad, xl16, Wl2, Wfp, bl2r, bfm_base):
    grid = NPAD // _TCB

    def body(degp_ref, degl_ref, xp_ref, xl_ref, wl2_ref, wfp_ref, bl2_ref,
             bfm_ref, u1p_ref, u1l_ref, invp_ref, invl_ref, wfold_ref, bfmo_ref):
        invp = lax.rsqrt(degp_ref[:, 0:1] + 1.0)
        invl = lax.rsqrt(degl_ref[:, 0:1] + 1.0)
        u1p_ref[...] = xp_ref[...] * invp
        u1l_ref[...] = jnp.zeros_like(u1l_ref)
        u1l_ref[:, 0:16] = xl_ref[...] * invl
        invp_ref[...] = jnp.broadcast_to(invp, invp_ref.shape)
        invl_ref[...] = jnp.broadcast_to(invl, invl_ref.shape)

        @pl.when(pl.program_id(0) == 0)
        def _():
            wfold_ref[...] = jnp.dot(wl2_ref[...], wfp_ref[...],
                                     preferred_element_type=jnp.float32)
            bfmo_ref[...] = jnp.dot(bl2_ref[...], wfp_ref[...],
                                    preferred_element_type=jnp.float32) + bfm_ref[...]

    return pl.pallas_call(
        body,
        grid=(grid,),
        in_specs=[
            pl.BlockSpec((_TCB, 128), lambda i: (i, 0)),
            pl.BlockSpec((_TCB, 128), lambda i: (i, 0)),
            pl.BlockSpec((_TCB, 128), lambda i: (i, 0)),
            pl.BlockSpec((_TCB, 16), lambda i: (i, 0)),
            pl.BlockSpec((128, 128), lambda i: (0, 0)),
            pl.BlockSpec((128, 16), lambda i: (0, 0)),
            pl.BlockSpec((1, 128), lambda i: (0, 0)),
            pl.BlockSpec((1, 16), lambda i: (0, 0)),
        ],
        out_specs=[
            pl.BlockSpec((_TCB, 128), lambda i: (i, 0)),
            pl.BlockSpec((_TCB, 128), lambda i: (i, 0)),
            pl.BlockSpec((_TCB, 8), lambda i: (i, 0)),
            pl.BlockSpec((_TCB, 8), lambda i: (i, 0)),
            pl.BlockSpec((128, 16), lambda i: (0, 0)),
            pl.BlockSpec((1, 16), lambda i: (0, 0)),
        ],
        out_shape=[
            jax.ShapeDtypeStruct((NPAD, 128), jnp.float32),
            jax.ShapeDtypeStruct((NPAD, 128), jnp.float32),
            jax.ShapeDtypeStruct((NPAD, 8), jnp.float32),
            jax.ShapeDtypeStruct((NPAD, 8), jnp.float32),
            jax.ShapeDtypeStruct((128, 16), jnp.float32),
            jax.ShapeDtypeStruct((1, 16), jnp.float32),
        ],
    )(degp16, degl16, xp_pad, xl16, Wl2, Wfp, bl2r, bfm_base)


def _tc_layer1(Sp, u1, inv8, W1, b1r, kdim):
    grid = NPAD // _TCB

    def body(s0_ref, s1_ref, u1_ref, inv_ref, w_ref, b_ref, u2_ref):
        inv = inv_ref[:, 0:1]
        agg = inv * (s0_ref[...] + s1_ref[...] + u1_ref[...])
        h = jnp.maximum(jnp.dot(agg, w_ref[...],
                                preferred_element_type=jnp.float32) + b_ref[...], 0.0)
        u2_ref[...] = h * inv

    return pl.pallas_call(
        body,
        grid=(grid,),
        in_specs=[
            pl.BlockSpec((_TCB, kdim), lambda i: (i, 0)),
            pl.BlockSpec((_TCB, kdim), lambda i: (i, 0)),
            pl.BlockSpec((_TCB, kdim), lambda i: (i, 0)),
            pl.BlockSpec((_TCB, 8), lambda i: (i, 0)),
            pl.BlockSpec((kdim, 128), lambda i: (0, 0)),
            pl.BlockSpec((1, 128), lambda i: (0, 0)),
        ],
        out_specs=pl.BlockSpec((_TCB, 128), lambda i: (i, 0)),
        out_shape=jax.ShapeDtypeStruct((NPAD, 128), jnp.float32),
    )(Sp[0], Sp[1], u1, inv8, W1, b1r)


_TCB3 = 2000


def _tc_heads(S2p, u2p, invp8, S2l, u2l, invl8, labv16, wfold, bfm):
    grid = N // _TCB3

    def body(sp0_ref, sp1_ref, up_ref, ip_ref, sl0_ref, sl1_ref, ul_ref,
             il_ref, lab_ref, wf_ref, bfm_ref,
             saggp_ref, saggl_ref, logp_ref, slab_ref,
             accp, accl, acclp, acclab):
        i = pl.program_id(0)

        @pl.when(i == 0)
        def _():
            accp[...] = jnp.zeros_like(accp)
            accl[...] = jnp.zeros_like(accl)
            acclp[...] = jnp.zeros_like(acclp)
            acclab[...] = jnp.zeros_like(acclab)

        aggp = ip_ref[:, 0:1] * (sp0_ref[...] + sp1_ref[...] + up_ref[...])
        aggl = il_ref[:, 0:1] * (sl0_ref[...] + sl1_ref[...] + ul_ref[...])
        accp[...] += jnp.sum(aggp, axis=0, keepdims=True)
        accl[...] += jnp.sum(aggl, axis=0, keepdims=True)
        lab = lab_ref[...]
        acclab[...] += jnp.sum(lab, axis=0, keepdims=True)
        logits = jnp.dot(aggl, wf_ref[...],
                         preferred_element_type=jnp.float32) + bfm_ref[...]
        m = jnp.max(logits, axis=1, keepdims=True)
        e = jnp.exp(logits - m)
        probs = e / jnp.sum(e, axis=1, keepdims=True)
        p = jnp.sum(probs * lab, axis=1, keepdims=True)
        acclp[...] += jnp.sum(jnp.log(p), keepdims=True).reshape(1, 1)

        @pl.when(i == grid - 1)
        def _():
            saggp_ref[...] = accp[...]
            saggl_ref[...] = accl[...]
            logp_ref[...] = acclp[...]
            slab_ref[...] = acclab[...]

    return pl.pallas_call(
        body,
        grid=(grid,),
        in_specs=[
            pl.BlockSpec((_TCB3, 128), lambda i: (i, 0)),
            pl.BlockSpec((_TCB3, 128), lambda i: (i, 0)),
            pl.BlockSpec((_TCB3, 128), lambda i: (i, 0)),
            pl.BlockSpec((_TCB3, 8), lambda i: (i, 0)),
            pl.BlockSpec((_TCB3, 128), lambda i: (i, 0)),
            pl.BlockSpec((_TCB3, 128), lambda i: (i, 0)),
            pl.BlockSpec((_TCB3, 128), lambda i: (i, 0)),
            pl.BlockSpec((_TCB3, 8), lambda i: (i, 0)),
            pl.BlockSpec((_TCB3, 16), lambda i: (i, 0)),
            pl.BlockSpec((128, 16), lambda i: (0, 0)),
            pl.BlockSpec((1, 16), lambda i: (0, 0)),
        ],
        out_specs=[
            pl.BlockSpec((1, 128), lambda i: (0, 0)),
            pl.BlockSpec((1, 128), lambda i: (0, 0)),
            pl.BlockSpec((1, 1), lambda i: (0, 0)),
            pl.BlockSpec((1, 16), lambda i: (0, 0)),
        ],
        out_shape=[
            jax.ShapeDtypeStruct((1, 128), jnp.float32),
            jax.ShapeDtypeStruct((1, 128), jnp.float32),
            jax.ShapeDtypeStruct((1, 1), jnp.float32),
            jax.ShapeDtypeStruct((1, 16), jnp.float32),
        ],
        scratch_shapes=[
            pltpu.VMEM((1, 128), jnp.float32),
            pltpu.VMEM((1, 128), jnp.float32),
            pltpu.VMEM((1, 1), jnp.float32),
            pltpu.VMEM((1, 16), jnp.float32),
        ],
    )(S2p[0], S2p[1], u2p, invp8, S2l[0], S2l[1], u2l, invl8,
      labv16, wfold, bfm)


def _tc_decoder(s0d0, labv2, Wd1p, bd1r, Wd2, bd2r, Wp2, bp2r, Wl2, bl2r,
                saggp, saggl, slab):
    NV = float(N + 1)
    ISQ2 = 0.7071067811865476

    def body(idx_ref, lab_ref, wd1_ref, bd1_ref, wd2_ref, bd2_ref, wp2_ref,
             bp2_ref, wl2_ref, bl2_ref, saggp_ref, saggl_ref, slab_ref,
             zv_ref, zp_ref, hi_ref, ht_ref):
        s0 = idx_ref[0]
        d0 = idx_ref[1]
        rows = lax.broadcasted_iota(jnp.int32, (NPAD, 1), 0)
        oh = (rows == d0).astype(jnp.float32)
        oh_s = (rows == s0).astype(jnp.float32)
        valid = (rows < N + 1).astype(jnp.float32)
        inv = 1.0 - oh * (1.0 - ISQ2)
        lab = lab_ref[...]
        u = lab * inv
        u_s0 = jnp.sum(u * oh_s, axis=0, keepdims=True)
        agg1 = inv * (u + oh * u_s0)
        h = jnp.maximum(jnp.dot(agg1, wd1_ref[...],
                                preferred_element_type=jnp.float32) + bd1_ref[...], 0.0)
        u2 = h * inv
        u2_s0 = jnp.sum(u2 * oh_s, axis=0, keepdims=True)
        agg2 = inv * (u2 + oh * u2_s0)
        zv = jnp.dot(agg2, wd2_ref[...],
                     preferred_element_type=jnp.float32) + bd2_ref[...]
        zv_ref[...] = zv
        zv_mean = jnp.sum(zv * valid, axis=0, keepdims=True) / NV
        cols = lax.broadcasted_iota(jnp.int32, (1, 16), 1)
        lab2_mean = (slab_ref[...] + (cols == 10).astype(jnp.float32)) / NV
        ht_ref[...] = jnp.concatenate([zv_mean, lab2_mean], axis=1)
        mz_l = jnp.dot(saggl_ref[...] / float(N), wl2_ref[...],
                       preferred_element_type=jnp.float32) + bl2_ref[...]
        hi_ref[...] = jnp.concatenate([mz_l, slab_ref[...] / float(N)], axis=1)
        zp_ref[...] = jnp.dot(saggp_ref[...] / float(N), wp2_ref[...],
                              preferred_element_type=jnp.float32) + bp2_ref[...]

    return pl.pallas_call(
        body,
        grid=(1,),
        in_specs=[
            pl.BlockSpec(memory_space=pltpu.SMEM),
            pl.BlockSpec((NPAD, 16), lambda i: (0, 0)),
            pl.BlockSpec((16, 128), lambda i: (0, 0)),
            pl.BlockSpec((1, 128), lambda i: (0, 0)),
            pl.BlockSpec((128, 128), lambda i: (0, 0)),
            pl.BlockSpec((1, 128), lambda i: (0, 0)),
            pl.BlockSpec((128, 128), lambda i: (0, 0)),
            pl.BlockSpec((1, 128), lambda i: (0, 0)),
            pl.BlockSpec((128, 128), lambda i: (0, 0)),
            pl.BlockSpec((1, 128), lambda i: (0, 0)),
            pl.BlockSpec((1, 128), lambda i: (0, 0)),
            pl.BlockSpec((1, 128), lambda i: (0, 0)),
            pl.BlockSpec((1, 16), lambda i: (0, 0)),
        ],
        out_specs=[
            pl.BlockSpec((NPAD, 128), lambda i: (0, 0)),
            pl.BlockSpec((1, 128), lambda i: (0, 0)),
            pl.BlockSpec((1, 144), lambda i: (0, 0)),
            pl.BlockSpec((1, 144), lambda i: (0, 0)),
        ],
        out_shape=[
            jax.ShapeDtypeStruct((NPAD, 128), jnp.float32),
            jax.ShapeDtypeStruct((1, 128), jnp.float32),
            jax.ShapeDtypeStruct((1, 144), jnp.float32),
            jax.ShapeDtypeStruct((1, 144), jnp.float32),
        ],
    )(s0d0, labv2, Wd1p, bd1r, Wd2, bd2r, Wp2, bp2r, Wl2, bl2r, saggp, saggl, slab)


def _pad_edges(ei):
    pad = jnp.full((EPAD - E,), DUMP, jnp.int32)
    src = jnp.concatenate([ei[0].astype(jnp.int32), pad]).reshape(EROWS64, 64)
    dst = jnp.concatenate([ei[1].astype(jnp.int32), pad]).reshape(EROWS64, 64)
    return src, dst


def kernel(x_p, edge_index_p, x_l, edge_index_l, bfs_init, Wp1, bp1, Wp2, bp2,
           Wl1, bl1, Wl2, bl2, Wd1, bd1, Wd2, bd2, Wf, bf):
    f32 = jnp.float32
    srcp, dstp = _pad_edges(edge_index_p)
    srcl, dstl = _pad_edges(edge_index_l)
    dst_both = jnp.stack([dstp, dstl])
    ones128 = jnp.ones((64, 128), f32)
    zeros128 = jnp.zeros((RPT, 128), f32)

    xp_pad = jnp.pad(x_p, ((0, NPAD - N), (0, 0)))
    xl16 = jnp.pad(x_l, ((0, NPAD - N), (0, 1)))
    labv16 = jnp.pad(x_l[:, 4:], ((0, NPAD - N), (0, 5)))
    stop16 = jnp.zeros((1, 16), f32).at[0, 10].set(1.0)
    labv2 = jnp.concatenate(
        [labv16[:N], stop16, jnp.zeros((NPAD - N - 1, 16), f32)])

    Wl1p = jnp.pad(Wl1, ((0, 113), (0, 0)))
    Wd1p = jnp.pad(Wd1, ((0, 5), (0, 0)))
    Wfp = jnp.pad(Wf, ((0, 0), (0, 5)))
    bfm_base = (jnp.pad(bf, (0, 5)).reshape(1, 16)
                + jnp.concatenate([jnp.zeros((10,), f32),
                                   jnp.full((6,), -1e9, f32)]).reshape(1, 16))

    deg = _sc_deg_both(dst_both, ones128, zeros128)

    u1p, u1l, invp8, invl8, wfold, bfm = _tc_prep(
        deg[0], deg[1], xp_pad, xl16, Wl2, Wfp, bl2.reshape(1, 128), bfm_base)

    S1p = _sc_scatter_partial(u1p, srcp, dstp, zeros128, 128)
    S1l = _sc_scatter_partial(u1l, srcl, dstl, zeros128, 128)
    u2p = _tc_layer1(S1p, u1p, invp8, Wp1, bp1.reshape(1, 128), 128)
    u2l = _tc_layer1(S1l, u1l, invl8, Wl1p, bl1.reshape(1, 128), 128)

    S2p = _sc_scatter_partial(u2p, srcp, dstp, zeros128, 128)
    S2l = _sc_scatter_partial(u2l, srcl, dstl, zeros128, 128)
    saggp, saggl, logp, slab = _tc_heads(
        S2p, u2p, invp8, S2l, u2l, invl8, labv16, wfold, bfm)

    zv_pad, zp, hi, ht = _tc_decoder(
        bfs_init.reshape(2).astype(jnp.int32), labv2, Wd1p,
        bd1.reshape(1, 128), Wd2, bd2.reshape(1, 128), Wp2,
        bp2.reshape(1, 128), Wl2, bl2.reshape(1, 128), saggp, saggl, slab)

    log_prob = logp.reshape(())
    z_pocket = zp.reshape(128)
    z_v = zv_pad[:N + 1]
    H_init = hi.reshape(144)[:139]
    H_t = ht.reshape(144)[:139]
    return (log_prob, z_pocket, z_v, H_init, H_t)


# speedup vs baseline: 10.0430x; 1.0028x over previous
import functools

import jax
import jax.numpy as jnp
from jax import lax
from jax.experimental import pallas as pl
from jax.experimental.pallas import tpu as pltpu
from jax.experimental.pallas import tpu_sc as plsc

N = 10000
NPAD = 10240
E = 320000
EPAD = 327680
EROWS = 2560
DUMP = 10008
RPT = NPAD // 16
QD = 4
EROWS64 = 5120
EROWSA = 5152
SPLIT0 = 304
SPLIT1 = 16


def _sc_mesh():
    return plsc.VectorSubcoreMesh(core_axis_name="c", subcore_axis_name="s")


def _sc_deg_both(dst_both, ones128, zeros128):
    rows_per_tile = EROWS64 // 16

    @functools.partial(
        pl.kernel,
        out_type=jax.ShapeDtypeStruct((2, NPAD, 128), jnp.float32),
        mesh=_sc_mesh(),
        scratch_types=[
            pltpu.VMEM((rows_per_tile, 64), jnp.int32),
            pltpu.VMEM((64, 128), jnp.float32),
            pltpu.VMEM_SHARED((NPAD, 128), jnp.float32),
            pltpu.SemaphoreType.DMA,
        ],
    )
    def k(dst_hbm, ones_hbm, zeros_hbm, out_hbm, dst_v, ones_v, acc, ssem):
        c = lax.axis_index("c")
        s = lax.axis_index("s")
        pltpu.sync_copy(zeros_hbm, acc.at[pl.ds(s * RPT, RPT), :])
        pltpu.sync_copy(ones_hbm, ones_v)
        pltpu.sync_copy(dst_hbm.at[c, pl.ds(s * rows_per_tile, rows_per_tile), :], dst_v)
        plsc.subcore_barrier()

        def group(g, carry):
            j0 = g * QD
            sd = [pltpu.async_copy(ones_v, acc.at[dst_v.at[j0 + q]], ssem,
                                   add=True)
                  for q in range(QD)]
            for q in range(QD):
                sd[q].wait()
            return carry

        lax.fori_loop(0, rows_per_tile // QD, group, 0)
        plsc.subcore_barrier()
        pltpu.sync_copy(acc.at[pl.ds(s * RPT, RPT), :],
                        out_hbm.at[c, pl.ds(s * RPT, RPT), :])

    return k(dst_both, ones128, zeros128)


def _sc_scatter_partial(u_pad, src2d, dst2d, zeros, w):
    SG = 8 * QD

    @functools.partial(
        pl.kernel,
        out_type=jax.ShapeDtypeStruct((2, NPAD, w), jnp.float32),
        mesh=_sc_mesh(),
        scratch_types=[
            pltpu.VMEM((SG, 64), jnp.int32),
            pltpu.VMEM((SG, 64), jnp.int32),
            *[pltpu.VMEM((64, w), jnp.float32) for _ in range(QD)],
            pltpu.VMEM_SHARED((NPAD, w), jnp.float32),
            pltpu.SemaphoreType.DMA,
            pltpu.SemaphoreType.DMA,
        ],
    )
    def k(u_hbm, src_hbm, dst_hbm, zeros_hbm, out_hbm, src_v, dst_v,
          b0, b1, b2, b3, acc, gsem, ssem):
        c = lax.axis_index("c")
        s = lax.axis_index("s")
        base = jnp.where(c == 0, s * SPLIT0, 16 * SPLIT0 + s * SPLIT1)
        n_groups = jnp.where(c == 0, SPLIT0 // QD, SPLIT1 // QD)
        pltpu.sync_copy(zeros_hbm, acc.at[pl.ds(s * RPT, RPT), :])
        plsc.subcore_barrier()
        bufs = [b0, b1, b2, b3][:QD]

        def group(g, carry):
            @pl.when(g % (SG // QD) == 0)
            def _():
                sgb = base + (g // (SG // QD)) * SG
                pltpu.sync_copy(src_hbm.at[pl.ds(sgb, SG), :], src_v)
                pltpu.sync_copy(dst_hbm.at[pl.ds(sgb, SG), :], dst_v)

            lr = (g % (SG // QD)) * QD
            gd = [pltpu.async_copy(u_hbm.at[src_v.at[lr + q]], bufs[q], gsem)
                  for q in range(QD)]
            for q in range(QD):
                gd[q].wait()
            sd = [pltpu.async_copy(bufs[q], acc.at[dst_v.at[lr + q]], ssem,
                                   add=True)
                  for q in range(QD)]
            for q in range(QD):
                sd[q].wait()
            return carry

        lax.fori_loop(0, n_groups, group, 0)
        plsc.subcore_barrier()
        pltpu.sync_copy(acc.at[pl.ds(s * RPT, RPT), :],
                        out_hbm.at[c, pl.ds(s * RPT, RPT), :])

    return k(u_pad, src2d, dst2d, zeros)


_TCB = 2560


def _tc_prep(degp16, degl16, xp_pad, xl16, Wl2, Wfp, bl2r, bfm_base):
    grid = NPAD // _TCB

    def body(degp_ref, degl_ref, xp_ref, xl_ref, wl2_ref, wfp_ref, bl2_ref,
             bfm_ref, u1p_ref, u1l_ref, invp_ref, invl_ref, wfold_ref, bfmo_ref):
        invp = lax.rsqrt(degp_ref[:, 0:1] + 1.0)
        invl = lax.rsqrt(degl_ref[:, 0:1] + 1.0)
        u1p_ref[...] = xp_ref[...] * invp
        u1l_ref[...] = jnp.zeros_like(u1l_ref)
        u1l_ref[:, 0:16] = xl_ref[...] * invl
        invp_ref[...] = jnp.broadcast_to(invp, invp_ref.shape)
        invl_ref[...] = jnp.broadcast_to(invl, invl_ref.shape)

        @pl.when(pl.program_id(0) == 0)
        def _():
            wfold_ref[...] = jnp.dot(wl2_ref[...], wfp_ref[...],
                                     preferred_element_type=jnp.float32)
            bfmo_ref[...] = jnp.dot(bl2_ref[...], wfp_ref[...],
                                    preferred_element_type=jnp.float32) + bfm_ref[...]

    return pl.pallas_call(
        body,
        grid=(grid,),
        in_specs=[
            pl.BlockSpec((_TCB, 128), lambda i: (i, 0)),
            pl.BlockSpec((_TCB, 128), lambda i: (i, 0)),
            pl.BlockSpec((_TCB, 128), lambda i: (i, 0)),
            pl.BlockSpec((_TCB, 16), lambda i: (i, 0)),
            pl.BlockSpec((128, 128), lambda i: (0, 0)),
            pl.BlockSpec((128, 16), lambda i: (0, 0)),
            pl.BlockSpec((1, 128), lambda i: (0, 0)),
            pl.BlockSpec((1, 16), lambda i: (0, 0)),
        ],
        out_specs=[
            pl.BlockSpec((_TCB, 128), lambda i: (i, 0)),
            pl.BlockSpec((_TCB, 128), lambda i: (i, 0)),
            pl.BlockSpec((_TCB, 8), lambda i: (i, 0)),
            pl.BlockSpec((_TCB, 8), lambda i: (i, 0)),
            pl.BlockSpec((128, 16), lambda i: (0, 0)),
            pl.BlockSpec((1, 16), lambda i: (0, 0)),
        ],
        out_shape=[
            jax.ShapeDtypeStruct((NPAD, 128), jnp.float32),
            jax.ShapeDtypeStruct((NPAD, 128), jnp.float32),
            jax.ShapeDtypeStruct((NPAD, 8), jnp.float32),
            jax.ShapeDtypeStruct((NPAD, 8), jnp.float32),
            jax.ShapeDtypeStruct((128, 16), jnp.float32),
            jax.ShapeDtypeStruct((1, 16), jnp.float32),
        ],
    )(degp16, degl16, xp_pad, xl16, Wl2, Wfp, bl2r, bfm_base)


def _tc_layer1(Sp, u1, inv8, W1, b1r, kdim):
    grid = NPAD // _TCB

    def body(s0_ref, s1_ref, u1_ref, inv_ref, w_ref, b_ref, u2_ref):
        inv = inv_ref[:, 0:1]
        agg = inv * (s0_ref[...] + s1_ref[...] + u1_ref[...])
        h = jnp.maximum(jnp.dot(agg, w_ref[...],
                                preferred_element_type=jnp.float32) + b_ref[...], 0.0)
        u2_ref[...] = h * inv

    return pl.pallas_call(
        body,
        grid=(grid,),
        in_specs=[
            pl.BlockSpec((_TCB, kdim), lambda i: (i, 0)),
            pl.BlockSpec((_TCB, kdim), lambda i: (i, 0)),
            pl.BlockSpec((_TCB, kdim), lambda i: (i, 0)),
            pl.BlockSpec((_TCB, 8), lambda i: (i, 0)),
            pl.BlockSpec((kdim, 128), lambda i: (0, 0)),
            pl.BlockSpec((1, 128), lambda i: (0, 0)),
        ],
        out_specs=pl.BlockSpec((_TCB, 128), lambda i: (i, 0)),
        out_shape=jax.ShapeDtypeStruct((NPAD, 128), jnp.float32),
    )(Sp[0], Sp[1], u1, inv8, W1, b1r)


_TCB3 = 2000


def _tc_heads(S2p, u2p, invp8, S2l, u2l, invl8, labv16, wfold, bfm):
    grid = N // _TCB3

    def body(sp0_ref, sp1_ref, up_ref, ip_ref, sl0_ref, sl1_ref, ul_ref,
             il_ref, lab_ref, wf_ref, bfm_ref,
             saggp_ref, saggl_ref, logp_ref, slab_ref,
             accp, accl, acclp, acclab):
        i = pl.program_id(0)

        @pl.when(i == 0)
        def _():
            accp[...] = jnp.zeros_like(accp)
            accl[...] = jnp.zeros_like(accl)
            acclp[...] = jnp.zeros_like(acclp)
            acclab[...] = jnp.zeros_like(acclab)

        aggp = ip_ref[:, 0:1] * (sp0_ref[...] + sp1_ref[...] + up_ref[...])
        aggl = il_ref[:, 0:1] * (sl0_ref[...] + sl1_ref[...] + ul_ref[...])
        accp[...] += jnp.sum(aggp, axis=0, keepdims=True)
        accl[...] += jnp.sum(aggl, axis=0, keepdims=True)
        lab = lab_ref[...]
        acclab[...] += jnp.sum(lab, axis=0, keepdims=True)
        logits = jnp.dot(aggl, wf_ref[...],
                         preferred_element_type=jnp.float32) + bfm_ref[...]
        m = jnp.max(logits, axis=1, keepdims=True)
        e = jnp.exp(logits - m)
        probs = e / jnp.sum(e, axis=1, keepdims=True)
        p = jnp.sum(probs * lab, axis=1, keepdims=True)
        acclp[...] += jnp.sum(jnp.log(p), keepdims=True).reshape(1, 1)

        @pl.when(i == grid - 1)
        def _():
            saggp_ref[...] = accp[...]
            saggl_ref[...] = accl[...]
            logp_ref[...] = acclp[...]
            slab_ref[...] = acclab[...]

    return pl.pallas_call(
        body,
        grid=(grid,),
        in_specs=[
            pl.BlockSpec((_TCB3, 128), lambda i: (i, 0)),
            pl.BlockSpec((_TCB3, 128), lambda i: (i, 0)),
            pl.BlockSpec((_TCB3, 128), lambda i: (i, 0)),
            pl.BlockSpec((_TCB3, 8), lambda i: (i, 0)),
            pl.BlockSpec((_TCB3, 128), lambda i: (i, 0)),
            pl.BlockSpec((_TCB3, 128), lambda i: (i, 0)),
            pl.BlockSpec((_TCB3, 128), lambda i: (i, 0)),
            pl.BlockSpec((_TCB3, 8), lambda i: (i, 0)),
            pl.BlockSpec((_TCB3, 16), lambda i: (i, 0)),
            pl.BlockSpec((128, 16), lambda i: (0, 0)),
            pl.BlockSpec((1, 16), lambda i: (0, 0)),
        ],
        out_specs=[
            pl.BlockSpec((1, 128), lambda i: (0, 0)),
            pl.BlockSpec((1, 128), lambda i: (0, 0)),
            pl.BlockSpec((1, 1), lambda i: (0, 0)),
            pl.BlockSpec((1, 16), lambda i: (0, 0)),
        ],
        out_shape=[
            jax.ShapeDtypeStruct((1, 128), jnp.float32),
            jax.ShapeDtypeStruct((1, 128), jnp.float32),
            jax.ShapeDtypeStruct((1, 1), jnp.float32),
            jax.ShapeDtypeStruct((1, 16), jnp.float32),
        ],
        scratch_shapes=[
            pltpu.VMEM((1, 128), jnp.float32),
            pltpu.VMEM((1, 128), jnp.float32),
            pltpu.VMEM((1, 1), jnp.float32),
            pltpu.VMEM((1, 16), jnp.float32),
        ],
    )(S2p[0], S2p[1], u2p, invp8, S2l[0], S2l[1], u2l, invl8,
      labv16, wfold, bfm)


def _tc_decoder(s0d0, labv2, Wd1p, bd1r, Wd2, bd2r, Wp2, bp2r, Wl2, bl2r,
                saggp, saggl, slab):
    NV = float(N + 1)
    ISQ2 = 0.7071067811865476

    def body(idx_ref, lab_ref, wd1_ref, bd1_ref, wd2_ref, bd2_ref, wp2_ref,
             bp2_ref, wl2_ref, bl2_ref, saggp_ref, saggl_ref, slab_ref,
             zv_ref, zp_ref, hi_ref, ht_ref):
        s0 = idx_ref[0]
        d0 = idx_ref[1]
        rows = lax.broadcasted_iota(jnp.int32, (NPAD, 1), 0)
        oh = (rows == d0).astype(jnp.float32)
        oh_s = (rows == s0).astype(jnp.float32)
        valid = (rows < N + 1).astype(jnp.float32)
        inv = 1.0 - oh * (1.0 - ISQ2)
        lab = lab_ref[...]
        u = lab * inv
        u_s0 = jnp.sum(u * oh_s, axis=0, keepdims=True)
        agg1 = inv * (u + oh * u_s0)
        h = jnp.maximum(jnp.dot(agg1, wd1_ref[...],
                                preferred_element_type=jnp.float32) + bd1_ref[...], 0.0)
        u2 = h * inv
        u2_s0 = jnp.sum(u2 * oh_s, axis=0, keepdims=True)
        agg2 = inv * (u2 + oh * u2_s0)
        zv = jnp.dot(agg2, wd2_ref[...],
                     preferred_element_type=jnp.float32) + bd2_ref[...]
        zv_ref[...] = zv
        zv_mean = jnp.sum(zv * valid, axis=0, keepdims=True) / NV
        cols = lax.broadcasted_iota(jnp.int32, (1, 16), 1)
        lab2_mean = (slab_ref[...] + (cols == 10).astype(jnp.float32)) / NV
        ht_ref[...] = jnp.concatenate([zv_mean, lab2_mean], axis=1)
        mz_l = jnp.dot(saggl_ref[...] / float(N), wl2_ref[...],
                       preferred_element_type=jnp.float32) + bl2_ref[...]
        hi_ref[...] = jnp.concatenate([mz_l, slab_ref[...] / float(N)], axis=1)
        zp_ref[...] = jnp.dot(saggp_ref[...] / float(N), wp2_ref[...],
                              preferred_element_type=jnp.float32) + bp2_ref[...]

    return pl.pallas_call(
        body,
        grid=(1,),
        in_specs=[
            pl.BlockSpec(memory_space=pltpu.SMEM),
            pl.BlockSpec((NPAD, 16), lambda i: (0, 0)),
            pl.BlockSpec((16, 128), lambda i: (0, 0)),
            pl.BlockSpec((1, 128), lambda i: (0, 0)),
            pl.BlockSpec((128, 128), lambda i: (0, 0)),
            pl.BlockSpec((1, 128), lambda i: (0, 0)),
            pl.BlockSpec((128, 128), lambda i: (0, 0)),
            pl.BlockSpec((1, 128), lambda i: (0, 0)),
            pl.BlockSpec((128, 128), lambda i: (0, 0)),
            pl.BlockSpec((1, 128), lambda i: (0, 0)),
            pl.BlockSpec((1, 128), lambda i: (0, 0)),
            pl.BlockSpec((1, 128), lambda i: (0, 0)),
            pl.BlockSpec((1, 16), lambda i: (0, 0)),
        ],
        out_specs=[
            pl.BlockSpec((NPAD, 128), lambda i: (0, 0)),
            pl.BlockSpec((1, 128), lambda i: (0, 0)),
            pl.BlockSpec((1, 144), lambda i: (0, 0)),
            pl.BlockSpec((1, 144), lambda i: (0, 0)),
        ],
        out_shape=[
            jax.ShapeDtypeStruct((NPAD, 128), jnp.float32),
            jax.ShapeDtypeStruct((1, 128), jnp.float32),
            jax.ShapeDtypeStruct((1, 144), jnp.float32),
            jax.ShapeDtypeStruct((1, 144), jnp.float32),
        ],
    )(s0d0, labv2, Wd1p, bd1r, Wd2, bd2r, Wp2, bp2r, Wl2, bl2r, saggp, saggl, slab)


def _pad_edges(ei):
    pad = jnp.full((EROWSA * 64 - E,), DUMP, jnp.int32)
    src = jnp.concatenate([ei[0].astype(jnp.int32), pad]).reshape(EROWSA, 64)
    dst = jnp.concatenate([ei[1].astype(jnp.int32), pad]).reshape(EROWSA, 64)
    return src, dst


def kernel(x_p, edge_index_p, x_l, edge_index_l, bfs_init, Wp1, bp1, Wp2, bp2,
           Wl1, bl1, Wl2, bl2, Wd1, bd1, Wd2, bd2, Wf, bf):
    f32 = jnp.float32
    srcp, dstp = _pad_edges(edge_index_p)
    srcl, dstl = _pad_edges(edge_index_l)
    dst_both = jnp.stack([dstp, dstl])
    ones128 = jnp.ones((64, 128), f32)
    zeros128 = jnp.zeros((RPT, 128), f32)

    xp_pad = jnp.pad(x_p, ((0, NPAD - N), (0, 0)))
    xl16 = jnp.pad(x_l, ((0, NPAD - N), (0, 1)))
    labv16 = jnp.pad(x_l[:, 4:], ((0, NPAD - N), (0, 5)))
    stop16 = jnp.zeros((1, 16), f32).at[0, 10].set(1.0)
    labv2 = jnp.concatenate(
        [labv16[:N], stop16, jnp.zeros((NPAD - N - 1, 16), f32)])

    Wl1p = jnp.pad(Wl1, ((0, 113), (0, 0)))
    Wd1p = jnp.pad(Wd1, ((0, 5), (0, 0)))
    Wfp = jnp.pad(Wf, ((0, 0), (0, 5)))
    bfm_base = (jnp.pad(bf, (0, 5)).reshape(1, 16)
                + jnp.concatenate([jnp.zeros((10,), f32),
                                   jnp.full((6,), -1e9, f32)]).reshape(1, 16))

    deg = _sc_deg_both(dst_both, ones128, zeros128)

    u1p, u1l, invp8, invl8, wfold, bfm = _tc_prep(
        deg[0], deg[1], xp_pad, xl16, Wl2, Wfp, bl2.reshape(1, 128), bfm_base)

    S1p = _sc_scatter_partial(u1p, srcp, dstp, zeros128, 128)
    S1l = _sc_scatter_partial(u1l, srcl, dstl, zeros128, 128)
    u2p = _tc_layer1(S1p, u1p, invp8, Wp1, bp1.reshape(1, 128), 128)
    u2l = _tc_layer1(S1l, u1l, invl8, Wl1p, bl1.reshape(1, 128), 128)

    S2p = _sc_scatter_partial(u2p, srcp, dstp, zeros128, 128)
    S2l = _sc_scatter_partial(u2l, srcl, dstl, zeros128, 128)
    saggp, saggl, logp, slab = _tc_heads(
        S2p, u2p, invp8, S2l, u2l, invl8, labv16, wfold, bfm)

    zv_pad, zp, hi, ht = _tc_decoder(
        bfs_init.reshape(2).astype(jnp.int32), labv2, Wd1p,
        bd1.reshape(1, 128), Wd2, bd2.reshape(1, 128), Wp2,
        bp2.reshape(1, 128), Wl2, bl2.reshape(1, 128), saggp, saggl, slab)

    log_prob = logp.reshape(())
    z_pocket = zp.reshape(128)
    z_v = zv_pad[:N + 1]
    H_init = hi.reshape(144)[:139]
    H_t = ht.reshape(144)[:139]
    return (log_prob, z_pocket, z_v, H_init, H_t)
